# Initial kernel scaffold; baseline (speedup 1.0000x reference)
import jax
import jax.numpy as jnp
from jax.experimental import pallas as pl


def kernel(x, W):
    raise NotImplementedError("write your pallas kernel here")


# speedup vs baseline: 3.7445x; 3.7445x over previous
import functools

import jax
import jax.numpy as jnp
from jax.experimental import pallas as pl
from jax.experimental.pallas import tpu as pltpu

DIM = 768
TOPK = 2
N_EXPERT = 64
ROUTE_SCALE = 1.0
ALPHA = 0.1

BLOCK_ROWS = 1024


def _gate_kernel(x_ref, wt_ref, w_out_ref, i_out_ref, aux_ref, pi_acc, fi_acc,
                 *, nblocks, blocks_per_batch, nbatch, n_tokens):
    i = pl.program_id(0)

    @pl.when(i == 0)
    def _init():
        pi_acc[:] = jnp.zeros_like(pi_acc)
        fi_acc[:] = jnp.zeros_like(fi_acc)

    s = jnp.dot(x_ref[:], wt_ref[:], preferred_element_type=jnp.float32)
    m = jnp.max(s, axis=-1, keepdims=True)
    e = jnp.exp(s - m)
    p = e / jnp.sum(e, axis=-1, keepdims=True)

    lane = jax.lax.broadcasted_iota(jnp.int32, p.shape, 1)
    a1 = jnp.argmax(p, axis=-1)
    v1 = jnp.max(p, axis=-1)
    m1 = lane == a1[:, None]
    pm = jnp.where(m1, -jnp.inf, p)
    a2 = jnp.argmax(pm, axis=-1)
    v2 = jnp.max(pm, axis=-1)
    m2 = lane == a2[:, None]

    w_out_ref[:] = jnp.stack([v1, v2], axis=1) * ROUTE_SCALE
    i_out_ref[:] = jnp.stack([a1, a2], axis=1)

    batch = i // blocks_per_batch
    bh = (jax.lax.broadcasted_iota(jnp.int32, (nbatch, 1), 0) == batch
          ).astype(jnp.float32)
    pi_acc[:] += bh * jnp.sum(p, axis=0)[None, :]
    cnt = jnp.sum(m1.astype(jnp.float32) + m2.astype(jnp.float32), axis=0)
    fi_acc[:] += bh * cnt[None, :]

    @pl.when(i == nblocks - 1)
    def _finish():
        fi = fi_acc[:] / (TOPK * n_tokens / N_EXPERT)
        pi = pi_acc[:] / n_tokens
        aux_ref[:, :] = jnp.sum(fi * pi, keepdims=True) * (ALPHA / nbatch)


def kernel(x, W):
    b, n, d = x.shape
    xf = x.reshape(-1, d)
    rows = b * n
    nblocks = rows // BLOCK_ROWS
    blocks_per_batch = n // BLOCK_ROWS
    wt = W.T

    body = functools.partial(
        _gate_kernel, nblocks=nblocks, blocks_per_batch=blocks_per_batch,
        nbatch=b, n_tokens=n)

    weight, idx, aux = pl.pallas_call(
        body,
        grid=(nblocks,),
        in_specs=[
            pl.BlockSpec((BLOCK_ROWS, d), lambda i: (i, 0)),
            pl.BlockSpec((d, N_EXPERT), lambda i: (0, 0)),
        ],
        out_specs=[
            pl.BlockSpec((BLOCK_ROWS, TOPK), lambda i: (i, 0)),
            pl.BlockSpec((BLOCK_ROWS, TOPK), lambda i: (i, 0)),
            pl.BlockSpec((1, 1), lambda i: (0, 0)),
        ],
        out_shape=[
            jax.ShapeDtypeStruct((rows, TOPK), jnp.float32),
            jax.ShapeDtypeStruct((rows, TOPK), jnp.int32),
            jax.ShapeDtypeStruct((1, 1), jnp.float32),
        ],
        scratch_shapes=[
            pltpu.VMEM((b, N_EXPERT), jnp.float32),
            pltpu.VMEM((b, N_EXPERT), jnp.float32),
        ],
    )(xf, wt)
    return weight, idx, aux[0, 0]


# speedup vs baseline: 4.2848x; 1.1443x over previous
import functools

import jax
import jax.numpy as jnp
from jax.experimental import pallas as pl
from jax.experimental.pallas import tpu as pltpu

DIM = 768
TOPK = 2
N_EXPERT = 64
ROUTE_SCALE = 1.0
ALPHA = 0.1

BLOCK_ROWS = 2048


def _gate_kernel(x_ref, wt_ref, w_out_ref, i_out_ref, aux_ref, pi_acc, fi_acc,
                 *, nblocks, blocks_per_batch, nbatch, n_tokens):
    i = pl.program_id(0)

    @pl.when(i == 0)
    def _init():
        pi_acc[:] = jnp.zeros_like(pi_acc)
        fi_acc[:] = jnp.zeros_like(fi_acc)

    s = jnp.dot(x_ref[:], wt_ref[:], preferred_element_type=jnp.float32)
    m = jnp.max(s, axis=-1, keepdims=True)
    e = jnp.exp(s - m)
    p = e / jnp.sum(e, axis=-1, keepdims=True)

    lane = jax.lax.broadcasted_iota(jnp.int32, p.shape, 1)
    a1 = jnp.argmax(p, axis=-1)
    v1 = jnp.max(p, axis=-1)
    m1 = lane == a1[:, None]
    pm = jnp.where(m1, -jnp.inf, p)
    a2 = jnp.argmax(pm, axis=-1)
    v2 = jnp.max(pm, axis=-1)
    m2 = lane == a2[:, None]

    w_out_ref[:] = jnp.stack([v1, v2], axis=1) * ROUTE_SCALE
    i_out_ref[:] = jnp.stack([a1, a2], axis=1)

    batch = i // blocks_per_batch
    bh = (jax.lax.broadcasted_iota(jnp.int32, (nbatch, 1), 0) == batch
          ).astype(jnp.float32)
    pi_acc[:] += bh * jnp.sum(p, axis=0)[None, :]
    cnt = jnp.sum(m1.astype(jnp.float32) + m2.astype(jnp.float32), axis=0)
    fi_acc[:] += bh * cnt[None, :]

    @pl.when(i == nblocks - 1)
    def _finish():
        fi = fi_acc[:] / (TOPK * n_tokens / N_EXPERT)
        pi = pi_acc[:] / n_tokens
        aux_ref[:, :] = jnp.sum(fi * pi, keepdims=True) * (ALPHA / nbatch)


def kernel(x, W):
    b, n, d = x.shape
    xf = x.reshape(-1, d)
    rows = b * n
    nblocks = rows // BLOCK_ROWS
    blocks_per_batch = n // BLOCK_ROWS
    wt = W.T

    body = functools.partial(
        _gate_kernel, nblocks=nblocks, blocks_per_batch=blocks_per_batch,
        nbatch=b, n_tokens=n)

    weight, idx, aux = pl.pallas_call(
        body,
        grid=(nblocks,),
        in_specs=[
            pl.BlockSpec((BLOCK_ROWS, d), lambda i: (i, 0)),
            pl.BlockSpec((d, N_EXPERT), lambda i: (0, 0)),
        ],
        out_specs=[
            pl.BlockSpec((BLOCK_ROWS, TOPK), lambda i: (i, 0)),
            pl.BlockSpec((BLOCK_ROWS, TOPK), lambda i: (i, 0)),
            pl.BlockSpec((1, 1), lambda i: (0, 0)),
        ],
        out_shape=[
            jax.ShapeDtypeStruct((rows, TOPK), jnp.float32),
            jax.ShapeDtypeStruct((rows, TOPK), jnp.int32),
            jax.ShapeDtypeStruct((1, 1), jnp.float32),
        ],
        scratch_shapes=[
            pltpu.VMEM((b, N_EXPERT), jnp.float32),
            pltpu.VMEM((b, N_EXPERT), jnp.float32),
        ],
    )(xf, wt)
    return weight, idx, aux[0, 0]


# speedup vs baseline: 4.4509x; 1.0388x over previous
import functools

import jax
import jax.numpy as jnp
from jax.experimental import pallas as pl
from jax.experimental.pallas import tpu as pltpu

DIM = 768
TOPK = 2
N_EXPERT = 64
ROUTE_SCALE = 1.0
ALPHA = 0.1

BLOCK_ROWS = 4096


def _gate_kernel(x_ref, wt_ref, w_out_ref, i_out_ref, aux_ref, pi_acc, fi_acc,
                 *, nblocks, blocks_per_batch, nbatch, n_tokens):
    i = pl.program_id(0)

    @pl.when(i == 0)
    def _init():
        pi_acc[:] = jnp.zeros_like(pi_acc)
        fi_acc[:] = jnp.zeros_like(fi_acc)

    s = jnp.dot(x_ref[:], wt_ref[:], preferred_element_type=jnp.float32)
    m = jnp.max(s, axis=-1, keepdims=True)
    e = jnp.exp(s - m)
    p = e / jnp.sum(e, axis=-1, keepdims=True)

    lane = jax.lax.broadcasted_iota(jnp.int32, p.shape, 1)
    a1 = jnp.argmax(p, axis=-1)
    v1 = jnp.max(p, axis=-1)
    m1 = lane == a1[:, None]
    pm = jnp.where(m1, -jnp.inf, p)
    a2 = jnp.argmax(pm, axis=-1)
    v2 = jnp.max(pm, axis=-1)
    m2 = lane == a2[:, None]

    w_out_ref[:] = jnp.stack([v1, v2], axis=1) * ROUTE_SCALE
    i_out_ref[:] = jnp.stack([a1, a2], axis=1)

    batch = i // blocks_per_batch
    bh = (jax.lax.broadcasted_iota(jnp.int32, (nbatch, 1), 0) == batch
          ).astype(jnp.float32)
    pi_acc[:] += bh * jnp.sum(p, axis=0)[None, :]
    cnt = jnp.sum(m1.astype(jnp.float32) + m2.astype(jnp.float32), axis=0)
    fi_acc[:] += bh * cnt[None, :]

    @pl.when(i == nblocks - 1)
    def _finish():
        fi = fi_acc[:] / (TOPK * n_tokens / N_EXPERT)
        pi = pi_acc[:] / n_tokens
        aux_ref[:, :] = jnp.sum(fi * pi, keepdims=True) * (ALPHA / nbatch)


def kernel(x, W):
    b, n, d = x.shape
    xf = x.reshape(-1, d)
    rows = b * n
    nblocks = rows // BLOCK_ROWS
    blocks_per_batch = n // BLOCK_ROWS
    wt = W.T

    body = functools.partial(
        _gate_kernel, nblocks=nblocks, blocks_per_batch=blocks_per_batch,
        nbatch=b, n_tokens=n)

    weight, idx, aux = pl.pallas_call(
        body,
        grid=(nblocks,),
        in_specs=[
            pl.BlockSpec((BLOCK_ROWS, d), lambda i: (i, 0)),
            pl.BlockSpec((d, N_EXPERT), lambda i: (0, 0)),
        ],
        out_specs=[
            pl.BlockSpec((BLOCK_ROWS, TOPK), lambda i: (i, 0)),
            pl.BlockSpec((BLOCK_ROWS, TOPK), lambda i: (i, 0)),
            pl.BlockSpec((1, 1), lambda i: (0, 0)),
        ],
        out_shape=[
            jax.ShapeDtypeStruct((rows, TOPK), jnp.float32),
            jax.ShapeDtypeStruct((rows, TOPK), jnp.int32),
            jax.ShapeDtypeStruct((1, 1), jnp.float32),
        ],
        scratch_shapes=[
            pltpu.VMEM((b, N_EXPERT), jnp.float32),
            pltpu.VMEM((b, N_EXPERT), jnp.float32),
        ],
    )(xf, wt)
    return weight, idx, aux[0, 0]


# speedup vs baseline: 4.4908x; 1.0090x over previous
import functools

import jax
import jax.numpy as jnp
from jax.experimental import pallas as pl
from jax.experimental.pallas import tpu as pltpu

DIM = 768
TOPK = 2
N_EXPERT = 64
ROUTE_SCALE = 1.0
ALPHA = 0.1

BLOCK_ROWS = 512
NSTREAM = 8


def _gate_kernel(*refs, nsteps, rows_per_step, n_tokens, nbatch):
    x_refs = refs[:NSTREAM]
    wt_ref = refs[NSTREAM]
    w_out_ref, i_out_ref, aux_ref, pi_acc, fi_acc = refs[NSTREAM + 1:]
    i = pl.program_id(0)

    @pl.when(i == 0)
    def _init():
        pi_acc[:] = jnp.zeros_like(pi_acc)
        fi_acc[:] = jnp.zeros_like(fi_acc)

    psum = jnp.zeros((N_EXPERT,), jnp.float32)
    cnt = jnp.zeros((N_EXPERT,), jnp.float32)
    for j, xr in enumerate(x_refs):
        s = jnp.dot(xr[:], wt_ref[:], preferred_element_type=jnp.float32)
        m = jnp.max(s, axis=-1, keepdims=True)
        e = jnp.exp(s - m)
        p = e / jnp.sum(e, axis=-1, keepdims=True)

        lane = jax.lax.broadcasted_iota(jnp.int32, p.shape, 1)
        a1 = jnp.argmax(p, axis=-1)
        v1 = jnp.max(p, axis=-1)
        m1 = lane == a1[:, None]
        pm = jnp.where(m1, -jnp.inf, p)
        a2 = jnp.argmax(pm, axis=-1)
        v2 = jnp.max(pm, axis=-1)
        m2 = lane == a2[:, None]

        rows = pl.ds(j * BLOCK_ROWS, BLOCK_ROWS)
        w_out_ref[rows, :] = jnp.stack([v1, v2], axis=1) * ROUTE_SCALE
        i_out_ref[rows, :] = jnp.stack([a1, a2], axis=1)

        psum = psum + jnp.sum(p, axis=0)
        cnt = cnt + jnp.sum(
            jnp.where(m1 | m2, 1.0, 0.0).astype(jnp.float32), axis=0)

    batch = i // (n_tokens // rows_per_step)
    bh = (jax.lax.broadcasted_iota(jnp.int32, (nbatch, 1), 0) == batch
          ).astype(jnp.float32)
    pi_acc[:] += bh * psum[None, :]
    fi_acc[:] += bh * cnt[None, :]

    @pl.when(i == nsteps - 1)
    def _finish():
        fi = fi_acc[:] / (TOPK * n_tokens / N_EXPERT)
        pi = pi_acc[:] / n_tokens
        aux_ref[:, :] = jnp.sum(fi * pi, keepdims=True) * (ALPHA / nbatch)


def _mk_spec(j, d):
    return pl.BlockSpec((BLOCK_ROWS, d), lambda i, j=j: (NSTREAM * i + j, 0))


def kernel(x, W):
    b, n, d = x.shape
    xf = x.reshape(-1, d)
    rows = b * n
    rows_per_step = NSTREAM * BLOCK_ROWS
    nsteps = rows // rows_per_step
    wt = W.T

    body = functools.partial(
        _gate_kernel, nsteps=nsteps, rows_per_step=rows_per_step,
        n_tokens=n, nbatch=b)

    weight, idx, aux = pl.pallas_call(
        body,
        grid=(nsteps,),
        in_specs=[_mk_spec(j, d) for j in range(NSTREAM)] + [
            pl.BlockSpec((d, N_EXPERT), lambda i: (0, 0)),
        ],
        out_specs=[
            pl.BlockSpec((rows_per_step, TOPK), lambda i: (i, 0)),
            pl.BlockSpec((rows_per_step, TOPK), lambda i: (i, 0)),
            pl.BlockSpec((1, 1), lambda i: (0, 0)),
        ],
        out_shape=[
            jax.ShapeDtypeStruct((rows, TOPK), jnp.float32),
            jax.ShapeDtypeStruct((rows, TOPK), jnp.int32),
            jax.ShapeDtypeStruct((1, 1), jnp.float32),
        ],
        scratch_shapes=[
            pltpu.VMEM((b, N_EXPERT), jnp.float32),
            pltpu.VMEM((b, N_EXPERT), jnp.float32),
        ],
    )(*([xf] * NSTREAM), wt)
    return weight, idx, aux[0, 0]
